# baseline (speedup 1.0000x reference)
import jax
import jax.numpy as jnp
from jax.experimental import pallas as pl

N_VAR = 100000
N_CON = 100000


def _apply_lin(p, x):
    return x @ p["W"] + p["b"]


def _transformer_conv(p, x_src, x_dst, edge_attr, src_idx, dst_idx, n_dst, out_ch):
    q = _apply_lin(p["q"], x_dst)
    k = _apply_lin(p["k"], x_src)
    v = _apply_lin(p["v"], x_src)
    ee = _apply_lin(p["e"], edge_attr)
    k_e = k[src_idx] + ee
    alpha = jnp.sum(q[dst_idx] * k_e, axis=-1) / jnp.sqrt(jnp.float32(out_ch))
    amax = jax.ops.segment_max(alpha, dst_idx, num_segments=n_dst)
    amax = jnp.where(jnp.isfinite(amax), amax, 0.0)
    ex = jnp.exp(alpha - amax[dst_idx])
    denom = jax.ops.segment_sum(ex, dst_idx, num_segments=n_dst)
    alpha_n = ex / (denom[dst_idx] + 1e-16)
    msg = (v[src_idx] + ee) * alpha_n[:, None]
    agg = jax.ops.segment_sum(msg, dst_idx, num_segments=n_dst)
    cnt = jax.ops.segment_sum(jnp.ones_like(alpha), dst_idx, num_segments=n_dst)
    agg = agg / jnp.clip(cnt, 1.0, None)[:, None]
    return agg + _apply_lin(p["skip"], x_dst)


def _identity_kernel(x_ref, o_ref):
    o_ref[...] = x_ref[...]


def _pallas_identity(x):
    blk = 25000
    return pl.pallas_call(
        _identity_kernel,
        grid=(x.shape[0] // blk,),
        in_specs=[pl.BlockSpec((blk, x.shape[1]), lambda i: (i, 0))],
        out_specs=pl.BlockSpec((blk, x.shape[1]), lambda i: (i, 0)),
        out_shape=jax.ShapeDtypeStruct(x.shape, x.dtype),
    )(x)


def kernel(var_lp_f, con_lp_f, lo_costs, hi_costs, def_mm, edge_lp_f_wo_ss, params, edge_index_var_con):
    src = edge_index_var_con[0]
    dst = edge_index_var_con[1]
    var_comb = var_lp_f
    con_comb = con_lp_f
    edge_comb = jnp.concatenate(
        [lo_costs[:, None], hi_costs[:, None], def_mm[:, None], edge_lp_f_wo_ss], axis=1)
    con_learned = jax.nn.relu(_transformer_conv(
        params["con_conv"], var_comb, con_comb, edge_comb, src, dst, N_CON, 16))
    con_comb2 = jnp.concatenate([con_learned, con_lp_f], axis=1)
    var_learned = jax.nn.relu(_transformer_conv(
        params["var_conv"], con_comb2, var_comb, edge_comb, dst, src, N_VAR, 16))
    var_comb2 = jnp.concatenate([var_learned, var_lp_f], axis=1)
    vcomp = jax.nn.relu(_apply_lin(params["vc2"], jax.nn.relu(_apply_lin(params["vc1"], var_comb2))))
    ccomp = jax.nn.relu(_apply_lin(params["cc2"], jax.nn.relu(_apply_lin(params["cc1"], con_comb2))))
    e_in = jnp.concatenate([edge_comb, vcomp[src], ccomp[dst]], axis=1)
    e_out = _apply_lin(params["em2"], jax.nn.relu(_apply_lin(params["em1"], e_in)))
    edge_learned = jax.nn.relu(_pallas_identity(e_out))
    return (var_learned, con_learned, edge_learned)


# speedup vs baseline: 8.2450x; 8.2450x over previous
import functools

import jax
import jax.numpy as jnp
from jax import lax
from jax.experimental import pallas as pl
from jax.experimental.pallas import tpu as pltpu
from jax.experimental.pallas import tpu_sc as plsc

F32 = jnp.float32
I32 = jnp.int32

NV = 100000
E = 1600000
NCORES = 2
NSUB = 16
NW = NCORES * NSUB
EPT = E // NW
CH = 80
NFULL = EPT // CH
AUXR = NV // 8
RPS = NV // NSUB
ZR = 250
CPS = 6248

_MESH = plsc.VectorSubcoreMesh(
    core_axis_name="c", subcore_axis_name="s",
    num_cores=NCORES, num_subcores=NSUB)


def _conv_body(gidx_hbm, sidx_hbm, kv_hbm, q_hbm, ee_hbm, z16_hbm,
               agg_out, aux_out,
               agg_s, aux_s, gidx_v, sidx_v, aux_idx,
               kvrows, qrows, eerows, aux_stage, sem1, sem2):
    c = lax.axis_index("c")
    s = lax.axis_index("s")
    wid = s * NCORES + c

    def zero_body(i, _):
        off = s * RPS + i * ZR
        pltpu.sync_copy(z16_hbm, agg_s.at[pl.ds(off, ZR)])
        return _
    lax.fori_loop(0, RPS // ZR, zero_body, 0)

    def zero_aux(i, _):
        j = s + NSUB * i

        @pl.when(j < AUXR // ZR)
        def _do():
            pltpu.sync_copy(z16_hbm, aux_s.at[pl.ds(j * ZR, ZR)])
        return _
    lax.fori_loop(0, (AUXR // ZR + NSUB - 1) // NSUB, zero_aux, 0)
    plsc.subcore_barrier()

    base = wid * EPT
    iota = lax.iota(I32, 16)
    ones = jnp.ones((16,), F32)

    def do_groups():
        def group(g, _):
            rows = g * 16 + iota
            acc = jnp.zeros((16,), F32)
            ecs = []
            for cc in range(16):
                col = jnp.full((16,), cc, I32)
                qc = plsc.load_gather(qrows, [rows, col])
                kc = plsc.load_gather(kvrows, [rows, col])
                ec = plsc.load_gather(eerows, [rows, col])
                ecs.append(ec)
                acc = acc + qc * (kc + ec)
            ex = jnp.exp(acc * 0.25)
            for cc in range(16):
                col = jnp.full((16,), cc, I32)
                vc = plsc.load_gather(kvrows, [rows, jnp.full((16,), 16 + cc, I32)])
                plsc.store_scatter(eerows, [rows, col], (vc + ecs[cc]) * ex)
            sd = sidx_v[pl.ds(g * 16, 16)]
            aux_idx[pl.ds(g * 16, 16)] = lax.shift_right_logical(sd, 3)
            colx = lax.shift_left(jnp.bitwise_and(sd, 7), 1)
            plsc.store_scatter(aux_stage, [rows, colx], ex)
            plsc.store_scatter(aux_stage, [rows, colx + 1], ones)
            return _
        lax.fori_loop(0, CH // 16, group, 0)

    def chunk_body(i, _):
        e0 = base + i * CH
        pltpu.sync_copy(gidx_hbm.at[pl.ds(e0, CH)], gidx_v)
        pltpu.sync_copy(sidx_hbm.at[pl.ds(e0, CH)], sidx_v)
        cp1 = pltpu.async_copy(kv_hbm.at[gidx_v], kvrows, sem1)
        cp2 = pltpu.async_copy(q_hbm.at[sidx_v], qrows, sem2)
        pltpu.sync_copy(ee_hbm.at[pl.ds(e0, CH)], eerows)
        pltpu.sync_copy(z16_hbm.at[pl.ds(0, CH)], aux_stage)
        cp1.wait()
        cp2.wait()
        do_groups()
        pltpu.sync_copy(eerows, agg_s.at[sidx_v], add=True)
        pltpu.sync_copy(aux_stage, aux_s.at[aux_idx], add=True)
        return _
    lax.fori_loop(0, NFULL, chunk_body, 0)

    plsc.subcore_barrier()
    off = s * CPS
    pltpu.sync_copy(agg_s.at[pl.ds(off, CPS)], agg_out.at[c, pl.ds(off, CPS)])

    @pl.when(s == NSUB - 1)
    def _tail():
        t0 = NSUB * CPS
        tn = NV - t0
        pltpu.sync_copy(agg_s.at[pl.ds(t0, tn)], agg_out.at[c, pl.ds(t0, tn)])

    @pl.when(s == 0)
    def _aux_dump():
        pltpu.sync_copy(aux_s, aux_out.at[c])


_conv_call = pl.kernel(
    _conv_body,
    out_type=[jax.ShapeDtypeStruct((NCORES, NV, 16), F32),
              jax.ShapeDtypeStruct((NCORES, AUXR, 16), F32)],
    mesh=_MESH,
    compiler_params=pltpu.CompilerParams(
        needs_layout_passes=False, use_tc_tiling_on_sc=False),
    scratch_types=[
        pltpu.VMEM_SHARED((NV, 16), F32),
        pltpu.VMEM_SHARED((AUXR, 16), F32),
        pltpu.VMEM((CH,), I32),
        pltpu.VMEM((CH,), I32),
        pltpu.VMEM((CH,), I32),
        pltpu.VMEM((CH, 32), F32),
        pltpu.VMEM((CH, 16), F32),
        pltpu.VMEM((CH, 16), F32),
        pltpu.VMEM((CH, 16), F32),
        pltpu.SemaphoreType.DMA,
        pltpu.SemaphoreType.DMA,
    ],
)


def _sc_conv(gidx, sidx, kv, q, ee):
    z16 = jnp.zeros((ZR, 16), F32)
    return _conv_call(gidx, sidx, kv, q, ee, z16)


def _gather_body(src_hbm, dst_hbm, vt_hbm, ct_hbm, gv_out, gc_out,
                 si_v, di_v, vrows, crows, sem1, sem2):
    c = lax.axis_index("c")
    s = lax.axis_index("s")
    wid = s * NCORES + c
    base = wid * EPT

    def chunk_body(i, _):
        e0 = base + i * CH
        pltpu.sync_copy(src_hbm.at[pl.ds(e0, CH)], si_v)
        pltpu.sync_copy(dst_hbm.at[pl.ds(e0, CH)], di_v)
        cp1 = pltpu.async_copy(vt_hbm.at[si_v], vrows, sem1)
        cp2 = pltpu.async_copy(ct_hbm.at[di_v], crows, sem2)
        cp1.wait()
        cp2.wait()
        pltpu.sync_copy(vrows, gv_out.at[pl.ds(e0, CH)])
        pltpu.sync_copy(crows, gc_out.at[pl.ds(e0, CH)])
        return _
    lax.fori_loop(0, NFULL, chunk_body, 0)


_gather_call = pl.kernel(
    _gather_body,
    out_type=[jax.ShapeDtypeStruct((E, 16), F32),
              jax.ShapeDtypeStruct((E, 16), F32)],
    mesh=_MESH,
    compiler_params=pltpu.CompilerParams(
        needs_layout_passes=False, use_tc_tiling_on_sc=False),
    scratch_types=[
        pltpu.VMEM((CH,), I32),
        pltpu.VMEM((CH,), I32),
        pltpu.VMEM((CH, 16), F32),
        pltpu.VMEM((CH, 16), F32),
        pltpu.SemaphoreType.DMA,
        pltpu.SemaphoreType.DMA,
    ],
)


def _full(shape):
    return pl.BlockSpec(shape, lambda i: tuple(0 for _ in shape))


def _rows(blk, w):
    return pl.BlockSpec((blk, w), lambda i: (i, 0))


def _prep1_body(con_ref, var_ref, wq, bq, wk, bk, wv, bv, ws, bs,
                q_ref, kv_ref, sk_ref):
    con = con_ref[...]
    var = var_ref[...]
    q_ref[...] = wq[...] @ con + bq[...]
    k = wk[...] @ var + bk[...]
    v = wv[...] @ var + bv[...]
    kv_ref[...] = jnp.concatenate([k, v], axis=0)
    sk_ref[...] = ws[...] @ con + bs[...]


def _cols(h, blk):
    return pl.BlockSpec((h, blk), lambda i: (0, i))


def _prep1(conT, varT, p):
    return pl.pallas_call(
        _prep1_body,
        out_shape=[jax.ShapeDtypeStruct((16, NV), F32),
                   jax.ShapeDtypeStruct((32, NV), F32),
                   jax.ShapeDtypeStruct((16, NV), F32)],
    )(conT, varT,
      p["q"]["W"].T, p["q"]["b"][:, None], p["k"]["W"].T, p["k"]["b"][:, None],
      p["v"]["W"].T, p["v"]["b"][:, None], p["skip"]["W"].T, p["skip"]["b"][:, None])


def _ee_body(ec_ref, w1, b1, w2, b2, e1_ref, e2_ref):
    ec = ec_ref[...]
    e1_ref[...] = w1[...] @ ec + b1[...]
    e2_ref[...] = w2[...] @ ec + b2[...]


def _ee(ecT, p1, p2):
    blk = 64000
    return pl.pallas_call(
        _ee_body,
        grid=(E // blk,),
        in_specs=[_cols(7, blk), _full((16, 7)), _full((16, 1)),
                  _full((16, 7)), _full((16, 1))],
        out_specs=[_cols(16, blk), _cols(16, blk)],
        out_shape=[jax.ShapeDtypeStruct((16, E), F32),
                   jax.ShapeDtypeStruct((16, E), F32)],
    )(ecT, p1["W"].T, p1["b"][:, None], p2["W"].T, p2["b"][:, None])


def _norm_agg(agg_ref, aux_ref, sk_ref):
    agg = agg_ref[0] + agg_ref[1]
    aux = aux_ref[0] + aux_ref[1]
    denom = aux[0:1, :]
    cnt = aux[1:2, :]
    return jax.nn.relu(
        agg / (denom + 1e-16) / jnp.clip(cnt, 1.0, None) + sk_ref[...])


def _combine1_body(agg_ref, aux_ref, sk_ref, con_ref,
                   wk, bk, wv, bv, c1w, c1b, c2w, c2b,
                   cl_ref, kv_ref, cc_ref):
    con_learned = _norm_agg(agg_ref, aux_ref, sk_ref)
    cl_ref[...] = con_learned
    comb = jnp.concatenate([con_learned, con_ref[...]], axis=0)
    k = wk[...] @ comb + bk[...]
    v = wv[...] @ comb + bv[...]
    kv_ref[...] = jnp.concatenate([k, v], axis=0)
    cc = jax.nn.relu(c1w[...] @ comb + c1b[...])
    cc_ref[...] = jax.nn.relu(c2w[...] @ cc + c2b[...])


def _combine1(aggT, auxT, sk1T, conT, pv, pcc1, pcc2):
    return pl.pallas_call(
        _combine1_body,
        out_shape=[jax.ShapeDtypeStruct((16, NV), F32),
                   jax.ShapeDtypeStruct((32, NV), F32),
                   jax.ShapeDtypeStruct((8, NV), F32)],
    )(aggT, auxT, sk1T, conT,
      pv["k"]["W"].T, pv["k"]["b"][:, None],
      pv["v"]["W"].T, pv["v"]["b"][:, None],
      pcc1["W"].T, pcc1["b"][:, None], pcc2["W"].T, pcc2["b"][:, None])


def _qs_body(x_ref, wq, bq, ws, bs, q_ref, sk_ref):
    x = x_ref[...]
    q_ref[...] = wq[...] @ x + bq[...]
    sk_ref[...] = ws[...] @ x + bs[...]


def _qs(xT, p):
    return pl.pallas_call(
        _qs_body,
        out_shape=[jax.ShapeDtypeStruct((16, NV), F32),
                   jax.ShapeDtypeStruct((16, NV), F32)],
    )(xT, p["q"]["W"].T, p["q"]["b"][:, None],
      p["skip"]["W"].T, p["skip"]["b"][:, None])


def _combine2_body(agg_ref, aux_ref, sk_ref, var_ref,
                   v1w, v1b, v2w, v2b, vl_ref, vc_ref):
    var_learned = _norm_agg(agg_ref, aux_ref, sk_ref)
    vl_ref[...] = var_learned
    comb = jnp.concatenate([var_learned, var_ref[...]], axis=0)
    vc = jax.nn.relu(v1w[...] @ comb + v1b[...])
    vc_ref[...] = jax.nn.relu(v2w[...] @ vc + v2b[...])


def _combine2(aggT, auxT, sk2T, varT, pvc1, pvc2):
    return pl.pallas_call(
        _combine2_body,
        out_shape=[jax.ShapeDtypeStruct((16, NV), F32),
                   jax.ShapeDtypeStruct((8, NV), F32)],
    )(aggT, auxT, sk2T, varT,
      pvc1["W"].T, pvc1["b"][:, None], pvc2["W"].T, pvc2["b"][:, None])


def _emlp_body(ec_ref, gv_ref, gc_ref, w1, b1, w2, b2, out_ref):
    e_in = jnp.concatenate([ec_ref[...], gv_ref[...], gc_ref[...]], axis=0)
    h = jax.nn.relu(w1[...] @ e_in + b1[...])
    out_ref[...] = jax.nn.relu(w2[...] @ h + b2[...])


def _emlp(ecT, gvT, gcT, p1, p2):
    blk = 64000
    return pl.pallas_call(
        _emlp_body,
        grid=(E // blk,),
        in_specs=[_cols(7, blk), _cols(8, blk), _cols(8, blk),
                  _full((8, 23)), _full((8, 1)), _full((8, 8)), _full((8, 1))],
        out_specs=_cols(8, blk),
        out_shape=jax.ShapeDtypeStruct((8, E), F32),
    )(ecT, gvT, gcT, p1["W"].T, p1["b"][:, None], p2["W"].T, p2["b"][:, None])


def kernel(var_lp_f, con_lp_f, lo_costs, hi_costs, def_mm, edge_lp_f_wo_ss,
           params, edge_index_var_con):
    src = edge_index_var_con[0]
    dst = edge_index_var_con[1]
    ecT = jnp.concatenate(
        [lo_costs[None], hi_costs[None], def_mm[None],
         edge_lp_f_wo_ss.T], axis=0)

    def unpack_aux(aux):
        return aux.reshape(2, AUXR * 8, 2).transpose(0, 2, 1)

    q1T, kv1T, sk1T = _prep1(con_lp_f.T, var_lp_f.T, params["con_conv"])
    ee1T, ee2T = _ee(ecT, params["con_conv"]["e"], params["var_conv"]["e"])

    agg1, aux1 = _sc_conv(src, dst, kv1T.T, q1T.T, ee1T.T)
    clT, kv2T, ccT = _combine1(
        agg1.transpose(0, 2, 1), unpack_aux(aux1), sk1T,
        con_lp_f.T, params["var_conv"], params["cc1"], params["cc2"])
    q2T, sk2T = _qs(var_lp_f.T, params["var_conv"])

    agg2, aux2 = _sc_conv(dst, src, kv2T.T, q2T.T, ee2T.T)
    vlT, vcT = _combine2(
        agg2.transpose(0, 2, 1), unpack_aux(aux2), sk2T,
        var_lp_f.T, params["vc1"], params["vc2"])

    pad8 = jnp.zeros((8, NV), F32)
    vtab = jnp.concatenate([vcT, pad8], axis=0).T
    ctab = jnp.concatenate([ccT, pad8], axis=0).T
    gv, gc = _gather_call(src, dst, vtab, ctab)
    elT = _emlp(ecT, gv.T[0:8], gc.T[0:8], params["em1"], params["em2"])
    return (vlT.T, clT.T, elT.T)


# speedup vs baseline: 8.6083x; 1.0441x over previous
import functools

import jax
import jax.numpy as jnp
from jax import lax
from jax.experimental import pallas as pl
from jax.experimental.pallas import tpu as pltpu
from jax.experimental.pallas import tpu_sc as plsc

F32 = jnp.float32
I32 = jnp.int32

NV = 100000
E = 1600000
NCORES = 2
NSUB = 16
NW = NCORES * NSUB
EPT = E // NW
CH = 80
NFULL = EPT // CH
AUXR = NV // 8
RPS = NV // NSUB
ZR = 250
CPS = 6248

_MESH = plsc.VectorSubcoreMesh(
    core_axis_name="c", subcore_axis_name="s",
    num_cores=NCORES, num_subcores=NSUB)


def _conv_body(gidx_hbm, sidx_hbm, kv_hbm, q_hbm, ee_hbm, z16_hbm,
               agg_out, aux_out,
               agg_s, aux_s,
               gidx0, sidx0, gidx1, sidx1,
               kv0, q0, ee0, kv1, q1, ee1,
               msg, aux_stage, aux_idx,
               semb0, semb1):
    c = lax.axis_index("c")
    s = lax.axis_index("s")
    wid = s * NCORES + c

    def zero_body(i, _):
        off = s * RPS + i * ZR
        pltpu.sync_copy(z16_hbm, agg_s.at[pl.ds(off, ZR)])
        return _
    lax.fori_loop(0, RPS // ZR, zero_body, 0)

    def zero_aux(i, _):
        j = s + NSUB * i

        @pl.when(j < AUXR // ZR)
        def _do():
            pltpu.sync_copy(z16_hbm, aux_s.at[pl.ds(j * ZR, ZR)])
        return _
    lax.fori_loop(0, (AUXR // ZR + NSUB - 1) // NSUB, zero_aux, 0)
    plsc.subcore_barrier()

    base = wid * EPT
    iota = lax.iota(I32, 16)
    ones = jnp.ones((16,), F32)
    zeros = jnp.zeros((16,), F32)

    def load_idx(i, gv, sv):
        e0 = base + i * CH
        pltpu.sync_copy(gidx_hbm.at[pl.ds(e0, CH)], gv)
        pltpu.sync_copy(sidx_hbm.at[pl.ds(e0, CH)], sv)

    def fire(i, gv, sv, kvb, qb, eeb, sem):
        e0 = base + i * CH
        pltpu.async_copy(kv_hbm.at[gv], kvb, sem)
        pltpu.async_copy(q_hbm.at[sv], qb, sem)
        pltpu.async_copy(ee_hbm.at[pl.ds(e0, CH)], eeb, sem)

    def drain(i, gv, sv, kvb, qb, eeb, sem):
        e0 = base + i * CH
        pltpu.make_async_copy(kv_hbm.at[gv], kvb, sem).wait()
        pltpu.make_async_copy(q_hbm.at[sv], qb, sem).wait()
        pltpu.make_async_copy(ee_hbm.at[pl.ds(e0, CH)], eeb, sem).wait()

    def compute_scatter(sv, kvb, qb, eeb):
        def group(g, _):
            rows = g * 16 + iota
            acc = jnp.zeros((16,), F32)
            ecs = []
            for cc in range(16):
                col = jnp.full((16,), cc, I32)
                qc = plsc.load_gather(qb, [rows, col])
                kc = plsc.load_gather(kvb, [rows, col])
                ec = plsc.load_gather(eeb, [rows, col])
                ecs.append(ec)
                acc = acc + qc * (kc + ec)
            ex = jnp.exp(acc * 0.25)
            for cc in range(16):
                col = jnp.full((16,), cc, I32)
                vc = plsc.load_gather(kvb, [rows, jnp.full((16,), 16 + cc, I32)])
                plsc.store_scatter(msg, [rows, col], (vc + ecs[cc]) * ex)
            for j in range(16):
                aux_stage[g * 16 + j] = zeros
            sd = sv[pl.ds(g * 16, 16)]
            aux_idx[pl.ds(g * 16, 16)] = lax.shift_right_logical(sd, 3)
            colx = lax.shift_left(jnp.bitwise_and(sd, 7), 1)
            plsc.store_scatter(aux_stage, [rows, colx], ex)
            plsc.store_scatter(aux_stage, [rows, colx + 1], ones)
            return _
        lax.fori_loop(0, CH // 16, group, 0)
        pltpu.sync_copy(msg, agg_s.at[sv], add=True)
        pltpu.sync_copy(aux_stage, aux_s.at[aux_idx], add=True)

    load_idx(0, gidx0, sidx0)
    fire(0, gidx0, sidx0, kv0, q0, ee0, semb0)

    def pair(t, _):
        i0 = 2 * t
        load_idx(i0 + 1, gidx1, sidx1)
        fire(i0 + 1, gidx1, sidx1, kv1, q1, ee1, semb1)
        drain(i0, gidx0, sidx0, kv0, q0, ee0, semb0)
        compute_scatter(sidx0, kv0, q0, ee0)
        load_idx(i0 + 2, gidx0, sidx0)
        fire(i0 + 2, gidx0, sidx0, kv0, q0, ee0, semb0)
        drain(i0 + 1, gidx1, sidx1, kv1, q1, ee1, semb1)
        compute_scatter(sidx1, kv1, q1, ee1)
        return _
    lax.fori_loop(0, (NFULL - 1) // 2, pair, 0)
    drain(NFULL - 1, gidx0, sidx0, kv0, q0, ee0, semb0)
    compute_scatter(sidx0, kv0, q0, ee0)

    plsc.subcore_barrier()
    off = s * CPS
    pltpu.sync_copy(agg_s.at[pl.ds(off, CPS)], agg_out.at[c, pl.ds(off, CPS)])

    @pl.when(s == NSUB - 1)
    def _tail():
        t0 = NSUB * CPS
        tn = NV - t0
        pltpu.sync_copy(agg_s.at[pl.ds(t0, tn)], agg_out.at[c, pl.ds(t0, tn)])

    @pl.when(s == 0)
    def _aux_dump():
        pltpu.sync_copy(aux_s, aux_out.at[c])


_conv_call = pl.kernel(
    _conv_body,
    out_type=[jax.ShapeDtypeStruct((NCORES, NV, 16), F32),
              jax.ShapeDtypeStruct((NCORES, AUXR, 16), F32)],
    mesh=_MESH,
    compiler_params=pltpu.CompilerParams(
        needs_layout_passes=False, use_tc_tiling_on_sc=False),
    scratch_types=[
        pltpu.VMEM_SHARED((NV, 16), F32),
        pltpu.VMEM_SHARED((AUXR, 16), F32),
        pltpu.VMEM((CH,), I32),
        pltpu.VMEM((CH,), I32),
        pltpu.VMEM((CH,), I32),
        pltpu.VMEM((CH,), I32),
        pltpu.VMEM((CH, 32), F32),
        pltpu.VMEM((CH, 16), F32),
        pltpu.VMEM((CH, 16), F32),
        pltpu.VMEM((CH, 32), F32),
        pltpu.VMEM((CH, 16), F32),
        pltpu.VMEM((CH, 16), F32),
        pltpu.VMEM((CH, 16), F32),
        pltpu.VMEM((CH, 16), F32),
        pltpu.VMEM((CH,), I32),
        pltpu.SemaphoreType.DMA,
        pltpu.SemaphoreType.DMA,
    ],
)


def _sc_conv(gidx, sidx, kv, q, ee):
    z16 = jnp.zeros((ZR, 16), F32)
    return _conv_call(gidx, sidx, kv, q, ee, z16)


def _gather_body(src_hbm, dst_hbm, vt_hbm, ct_hbm, gv_out, gc_out,
                 si0, di0, si1, di1, v0, c0, v1, c1, semb0, semb1):
    cx = lax.axis_index("c")
    s = lax.axis_index("s")
    wid = s * NCORES + cx
    base = wid * EPT

    def load_idx(i, sv, dv):
        e0 = base + i * CH
        pltpu.sync_copy(src_hbm.at[pl.ds(e0, CH)], sv)
        pltpu.sync_copy(dst_hbm.at[pl.ds(e0, CH)], dv)

    def fire(sv, dv, vb, cb, sem):
        pltpu.async_copy(vt_hbm.at[sv], vb, sem)
        pltpu.async_copy(ct_hbm.at[dv], cb, sem)

    def drain_store(i, sv, dv, vb, cb, sem):
        e0 = base + i * CH
        pltpu.make_async_copy(vt_hbm.at[sv], vb, sem).wait()
        pltpu.make_async_copy(ct_hbm.at[dv], cb, sem).wait()
        pltpu.sync_copy(vb, gv_out.at[pl.ds(e0, CH)])
        pltpu.sync_copy(cb, gc_out.at[pl.ds(e0, CH)])

    load_idx(0, si0, di0)
    fire(si0, di0, v0, c0, semb0)

    def pair(t, _):
        i0 = 2 * t
        load_idx(i0 + 1, si1, di1)
        fire(si1, di1, v1, c1, semb1)
        drain_store(i0, si0, di0, v0, c0, semb0)
        load_idx(i0 + 2, si0, di0)
        fire(si0, di0, v0, c0, semb0)
        drain_store(i0 + 1, si1, di1, v1, c1, semb1)
        return _
    lax.fori_loop(0, (NFULL - 1) // 2, pair, 0)
    drain_store(NFULL - 1, si0, di0, v0, c0, semb0)


_gather_call = pl.kernel(
    _gather_body,
    out_type=[jax.ShapeDtypeStruct((E, 16), F32),
              jax.ShapeDtypeStruct((E, 16), F32)],
    mesh=_MESH,
    compiler_params=pltpu.CompilerParams(
        needs_layout_passes=False, use_tc_tiling_on_sc=False),
    scratch_types=[
        pltpu.VMEM((CH,), I32),
        pltpu.VMEM((CH,), I32),
        pltpu.VMEM((CH,), I32),
        pltpu.VMEM((CH,), I32),
        pltpu.VMEM((CH, 16), F32),
        pltpu.VMEM((CH, 16), F32),
        pltpu.VMEM((CH, 16), F32),
        pltpu.VMEM((CH, 16), F32),
        pltpu.SemaphoreType.DMA,
        pltpu.SemaphoreType.DMA,
    ],
)


def _full(shape):
    return pl.BlockSpec(shape, lambda i: tuple(0 for _ in shape))


def _rows(blk, w):
    return pl.BlockSpec((blk, w), lambda i: (i, 0))


def _prep1_body(con_ref, var_ref, wq, bq, wk, bk, wv, bv, ws, bs,
                q_ref, kv_ref, sk_ref):
    con = con_ref[...]
    var = var_ref[...]
    q_ref[...] = wq[...] @ con + bq[...]
    k = wk[...] @ var + bk[...]
    v = wv[...] @ var + bv[...]
    kv_ref[...] = jnp.concatenate([k, v], axis=0)
    sk_ref[...] = ws[...] @ con + bs[...]


def _cols(h, blk):
    return pl.BlockSpec((h, blk), lambda i: (0, i))


def _prep1(conT, varT, p):
    return pl.pallas_call(
        _prep1_body,
        out_shape=[jax.ShapeDtypeStruct((16, NV), F32),
                   jax.ShapeDtypeStruct((32, NV), F32),
                   jax.ShapeDtypeStruct((16, NV), F32)],
    )(conT, varT,
      p["q"]["W"].T, p["q"]["b"][:, None], p["k"]["W"].T, p["k"]["b"][:, None],
      p["v"]["W"].T, p["v"]["b"][:, None], p["skip"]["W"].T, p["skip"]["b"][:, None])


def _ee_body(ec_ref, w1, b1, w2, b2, e1_ref, e2_ref):
    ec = ec_ref[...]
    e1_ref[...] = ec @ w1[...] + b1[...]
    e2_ref[...] = ec @ w2[...] + b2[...]


def _ee(edge_comb, p1, p2):
    blk = 12800
    return pl.pallas_call(
        _ee_body,
        grid=(E // blk,),
        in_specs=[_rows(blk, 7), _full((7, 16)), _full((1, 16)),
                  _full((7, 16)), _full((1, 16))],
        out_specs=[_rows(blk, 16), _rows(blk, 16)],
        out_shape=[jax.ShapeDtypeStruct((E, 16), F32),
                   jax.ShapeDtypeStruct((E, 16), F32)],
    )(edge_comb, p1["W"], p1["b"][None], p2["W"], p2["b"][None])


def _norm_agg(agg_ref, aux_ref, sk_ref):
    agg = agg_ref[0] + agg_ref[1]
    aux = aux_ref[0] + aux_ref[1]
    denom = aux[0:1, :]
    cnt = aux[1:2, :]
    return jax.nn.relu(
        agg / (denom + 1e-16) / jnp.clip(cnt, 1.0, None) + sk_ref[...])


def _combine1_body(agg_ref, aux_ref, sk_ref, con_ref,
                   wk, bk, wv, bv, c1w, c1b, c2w, c2b,
                   cl_ref, kv_ref, cc_ref):
    con_learned = _norm_agg(agg_ref, aux_ref, sk_ref)
    cl_ref[...] = con_learned
    comb = jnp.concatenate([con_learned, con_ref[...]], axis=0)
    k = wk[...] @ comb + bk[...]
    v = wv[...] @ comb + bv[...]
    kv_ref[...] = jnp.concatenate([k, v], axis=0)
    cc = jax.nn.relu(c1w[...] @ comb + c1b[...])
    cc_ref[...] = jax.nn.relu(c2w[...] @ cc + c2b[...])


def _combine1(aggT, auxT, sk1T, conT, pv, pcc1, pcc2):
    return pl.pallas_call(
        _combine1_body,
        out_shape=[jax.ShapeDtypeStruct((16, NV), F32),
                   jax.ShapeDtypeStruct((32, NV), F32),
                   jax.ShapeDtypeStruct((8, NV), F32)],
    )(aggT, auxT, sk1T, conT,
      pv["k"]["W"].T, pv["k"]["b"][:, None],
      pv["v"]["W"].T, pv["v"]["b"][:, None],
      pcc1["W"].T, pcc1["b"][:, None], pcc2["W"].T, pcc2["b"][:, None])


def _qs_body(x_ref, wq, bq, ws, bs, q_ref, sk_ref):
    x = x_ref[...]
    q_ref[...] = wq[...] @ x + bq[...]
    sk_ref[...] = ws[...] @ x + bs[...]


def _qs(xT, p):
    return pl.pallas_call(
        _qs_body,
        out_shape=[jax.ShapeDtypeStruct((16, NV), F32),
                   jax.ShapeDtypeStruct((16, NV), F32)],
    )(xT, p["q"]["W"].T, p["q"]["b"][:, None],
      p["skip"]["W"].T, p["skip"]["b"][:, None])


def _combine2_body(agg_ref, aux_ref, sk_ref, var_ref,
                   v1w, v1b, v2w, v2b, vl_ref, vc_ref):
    var_learned = _norm_agg(agg_ref, aux_ref, sk_ref)
    vl_ref[...] = var_learned
    comb = jnp.concatenate([var_learned, var_ref[...]], axis=0)
    vc = jax.nn.relu(v1w[...] @ comb + v1b[...])
    vc_ref[...] = jax.nn.relu(v2w[...] @ vc + v2b[...])


def _combine2(aggT, auxT, sk2T, varT, pvc1, pvc2):
    return pl.pallas_call(
        _combine2_body,
        out_shape=[jax.ShapeDtypeStruct((16, NV), F32),
                   jax.ShapeDtypeStruct((8, NV), F32)],
    )(aggT, auxT, sk2T, varT,
      pvc1["W"].T, pvc1["b"][:, None], pvc2["W"].T, pvc2["b"][:, None])


def _emlp_body(ec_ref, gv_ref, gc_ref, w1, b1, w2, b2, out_ref):
    e_in = jnp.concatenate(
        [ec_ref[...], gv_ref[:, 0:8], gc_ref[:, 0:8]], axis=1)
    h = jax.nn.relu(e_in @ w1[...] + b1[...])
    out_ref[...] = jax.nn.relu(h @ w2[...] + b2[...])


def _emlp(edge_comb, gv, gc, p1, p2):
    blk = 6400
    return pl.pallas_call(
        _emlp_body,
        grid=(E // blk,),
        in_specs=[_rows(blk, 7), _rows(blk, 16), _rows(blk, 16),
                  _full((23, 8)), _full((1, 8)), _full((8, 8)), _full((1, 8))],
        out_specs=_rows(blk, 8),
        out_shape=jax.ShapeDtypeStruct((E, 8), F32),
    )(edge_comb, gv, gc, p1["W"], p1["b"][None], p2["W"], p2["b"][None])


def kernel(var_lp_f, con_lp_f, lo_costs, hi_costs, def_mm, edge_lp_f_wo_ss,
           params, edge_index_var_con):
    src = edge_index_var_con[0]
    dst = edge_index_var_con[1]
    edge_comb = jnp.concatenate(
        [lo_costs[:, None], hi_costs[:, None], def_mm[:, None],
         edge_lp_f_wo_ss], axis=1)

    def unpack_aux(aux):
        return aux.reshape(2, AUXR * 8, 2).transpose(0, 2, 1)

    q1T, kv1T, sk1T = _prep1(con_lp_f.T, var_lp_f.T, params["con_conv"])
    ee1, ee2 = _ee(edge_comb, params["con_conv"]["e"], params["var_conv"]["e"])

    agg1, aux1 = _sc_conv(src, dst, kv1T.T, q1T.T, ee1)
    clT, kv2T, ccT = _combine1(
        agg1.transpose(0, 2, 1), unpack_aux(aux1), sk1T,
        con_lp_f.T, params["var_conv"], params["cc1"], params["cc2"])
    q2T, sk2T = _qs(var_lp_f.T, params["var_conv"])

    agg2, aux2 = _sc_conv(dst, src, kv2T.T, q2T.T, ee2)
    vlT, vcT = _combine2(
        agg2.transpose(0, 2, 1), unpack_aux(aux2), sk2T,
        var_lp_f.T, params["vc1"], params["vc2"])

    pad8 = jnp.zeros((8, NV), F32)
    vtab = jnp.concatenate([vcT, pad8], axis=0).T
    ctab = jnp.concatenate([ccT, pad8], axis=0).T
    gv, gc = _gather_call(src, dst, vtab, ctab)
    edge_learned = _emlp(edge_comb, gv, gc, params["em1"], params["em2"])
    return (vlT.T, clT.T, edge_learned)


# speedup vs baseline: 9.2715x; 1.0770x over previous
import functools

import jax
import jax.numpy as jnp
from jax import lax
from jax.experimental import pallas as pl
from jax.experimental.pallas import tpu as pltpu
from jax.experimental.pallas import tpu_sc as plsc

F32 = jnp.float32
I32 = jnp.int32

NV = 100000
E = 1600000
NCORES = 2
NSUB = 16
NW = NCORES * NSUB
EPT = E // NW
CH = 80
NFULL = EPT // CH
AUXR = NV // 8
RPS = NV // NSUB
ZR = 250
CPS = 6248

_MESH = plsc.VectorSubcoreMesh(
    core_axis_name="c", subcore_axis_name="s",
    num_cores=NCORES, num_subcores=NSUB)


def _conv_body(gidx_hbm, sidx_hbm, kv_hbm, q_hbm, ee_hbm, z16_hbm,
               agg_out, aux_out,
               agg_s, aux_s,
               gidx0, sidx0, gidx1, sidx1,
               kv0, q0, ee0, kv1, q1, ee1,
               msg0, stage0, aidx0, scidx0,
               msg1, stage1, aidx1, scidx1,
               semb0, semb1, semi0, semi1, sems0, sems1):
    c = lax.axis_index("c")
    s = lax.axis_index("s")
    wid = s * NCORES + c

    def zero_body(i, _):
        off = s * RPS + i * ZR
        pltpu.sync_copy(z16_hbm, agg_s.at[pl.ds(off, ZR)])
        return _
    lax.fori_loop(0, RPS // ZR, zero_body, 0)

    def zero_aux(i, _):
        j = s + NSUB * i

        @pl.when(j < AUXR // ZR)
        def _do():
            pltpu.sync_copy(z16_hbm, aux_s.at[pl.ds(j * ZR, ZR)])
        return _
    lax.fori_loop(0, (AUXR // ZR + NSUB - 1) // NSUB, zero_aux, 0)
    plsc.subcore_barrier()

    base = wid * EPT
    iota = lax.iota(I32, 16)
    ones = jnp.ones((16,), F32)
    zeros = jnp.zeros((16,), F32)

    def idx_copies(i, gv, sv, sem):
        e0 = base + i * CH
        return (pltpu.make_async_copy(gidx_hbm.at[pl.ds(e0, CH)], gv, sem),
                pltpu.make_async_copy(sidx_hbm.at[pl.ds(e0, CH)], sv, sem))

    def fire_idx(i, gv, sv, sem):
        for cp in idx_copies(i, gv, sv, sem):
            cp.start()

    def wait_idx(i, gv, sv, sem):
        for cp in idx_copies(i, gv, sv, sem):
            cp.wait()

    def gather_copies(i, gv, sv, kvb, qb, eeb, sem):
        e0 = base + i * CH
        return (pltpu.make_async_copy(kv_hbm.at[gv], kvb, sem),
                pltpu.make_async_copy(q_hbm.at[sv], qb, sem),
                pltpu.make_async_copy(ee_hbm.at[pl.ds(e0, CH)], eeb, sem))

    def fire_gathers(i, gv, sv, kvb, qb, eeb, sem):
        for cp in gather_copies(i, gv, sv, kvb, qb, eeb, sem):
            cp.start()

    def drain_gathers(i, gv, sv, kvb, qb, eeb, sem):
        for cp in gather_copies(i, gv, sv, kvb, qb, eeb, sem):
            cp.wait()

    def fire_scatters(msgb, stageb, aidxb, scidxb, sem):
        pltpu.async_copy(msgb, agg_s.at[scidxb], sem, add=True)
        pltpu.async_copy(stageb, aux_s.at[aidxb], sem, add=True)

    def drain_scatters(msgb, stageb, aidxb, scidxb, sem):
        pltpu.make_async_copy(msgb, agg_s.at[scidxb], sem).wait()
        pltpu.make_async_copy(stageb, aux_s.at[aidxb], sem).wait()

    def compute(sv, kvb, qb, eeb, msgb, stageb, aidxb, scidxb):
        def group(g, _):
            rows = g * 16 + iota
            acc = jnp.zeros((16,), F32)
            ecs = []
            for cc in range(16):
                col = jnp.full((16,), cc, I32)
                qc = plsc.load_gather(qb, [rows, col])
                kc = plsc.load_gather(kvb, [rows, col])
                ec = plsc.load_gather(eeb, [rows, col])
                ecs.append(ec)
                acc = acc + qc * (kc + ec)
            ex = jnp.exp(acc * 0.25)
            for cc in range(16):
                col = jnp.full((16,), cc, I32)
                vc = plsc.load_gather(kvb, [rows, jnp.full((16,), 16 + cc, I32)])
                plsc.store_scatter(msgb, [rows, col], (vc + ecs[cc]) * ex)
            for j in range(16):
                stageb[g * 16 + j] = zeros
            sd = sv[pl.ds(g * 16, 16)]
            scidxb[pl.ds(g * 16, 16)] = sd
            aidxb[pl.ds(g * 16, 16)] = lax.shift_right_logical(sd, 3)
            colx = lax.shift_left(jnp.bitwise_and(sd, 7), 1)
            plsc.store_scatter(stageb, [rows, colx], ex)
            plsc.store_scatter(stageb, [rows, colx + 1], ones)
            return _
        lax.fori_loop(0, CH // 16, group, 0)

    set0 = (gidx0, sidx0, kv0, q0, ee0, msg0, stage0, aidx0, scidx0,
            semb0, semi0, sems0)
    set1 = (gidx1, sidx1, kv1, q1, ee1, msg1, stage1, aidx1, scidx1,
            semb1, semi1, sems1)

    def half(t, i, cur, nxt, first, last):
        (gv, sv, kvb, qb, eeb, msgb, stageb, aidxb, scidxb,
         semb, semi, sems) = cur
        (ngv, nsv, nkvb, nqb, neeb, _nm, _ns, _na, _nc,
         nsemb, nsemi, _nss) = nxt
        wait_idx(i + 1, ngv, nsv, nsemi)
        fire_gathers(i + 1, ngv, nsv, nkvb, nqb, neeb, nsemb)
        drain_gathers(i, gv, sv, kvb, qb, eeb, semb)
        if not first:
            drain_scatters(msgb, stageb, aidxb, scidxb, sems)
        compute(sv, kvb, qb, eeb, msgb, stageb, aidxb, scidxb)
        fire_scatters(msgb, stageb, aidxb, scidxb, sems)
        if not last:
            fire_idx(i + 2, gv, sv, semi)

    def pair(t, _):
        i0 = 2 * t

        @pl.when(t == 0)
        def _first():
            half(t, i0, set0, set1, True, False)

        @pl.when(t > 0)
        def _rest():
            half(t, i0, set0, set1, False, False)

        @pl.when(t == 0)
        def _mid0():
            half(t, i0 + 1, set1, set0, True, False)

        @pl.when(jnp.logical_and(t > 0, t < (NFULL - 1) // 2 - 1))
        def _mid():
            half(t, i0 + 1, set1, set0, False, False)

        @pl.when(t == (NFULL - 1) // 2 - 1)
        def _lastpair():
            half(t, i0 + 1, set1, set0, False, True)
        return _

    gcp = idx_copies(0, gidx0, sidx0, semi0)
    for cp in gcp:
        cp.start()
    for cp in gcp:
        cp.wait()
    fire_gathers(0, gidx0, sidx0, kv0, q0, ee0, semb0)
    fire_idx(1, gidx1, sidx1, semi1)
    lax.fori_loop(0, (NFULL - 1) // 2, pair, 0)
    i = NFULL - 1
    drain_gathers(i, gidx0, sidx0, kv0, q0, ee0, semb0)
    drain_scatters(msg0, stage0, aidx0, scidx0, sems0)
    compute(sidx0, kv0, q0, ee0, msg0, stage0, aidx0, scidx0)
    fire_scatters(msg0, stage0, aidx0, scidx0, sems0)
    drain_scatters(msg0, stage0, aidx0, scidx0, sems0)
    drain_scatters(msg1, stage1, aidx1, scidx1, sems1)

    plsc.subcore_barrier()
    off = s * CPS
    pltpu.sync_copy(agg_s.at[pl.ds(off, CPS)], agg_out.at[c, pl.ds(off, CPS)])

    @pl.when(s == NSUB - 1)
    def _tail():
        t0 = NSUB * CPS
        tn = NV - t0
        pltpu.sync_copy(agg_s.at[pl.ds(t0, tn)], agg_out.at[c, pl.ds(t0, tn)])

    @pl.when(s == 0)
    def _aux_dump():
        pltpu.sync_copy(aux_s, aux_out.at[c])


_conv_call = pl.kernel(
    _conv_body,
    out_type=[jax.ShapeDtypeStruct((NCORES, NV, 16), F32),
              jax.ShapeDtypeStruct((NCORES, AUXR, 16), F32)],
    mesh=_MESH,
    compiler_params=pltpu.CompilerParams(
        needs_layout_passes=False, use_tc_tiling_on_sc=False),
    scratch_types=[
        pltpu.VMEM_SHARED((NV, 16), F32),
        pltpu.VMEM_SHARED((AUXR, 16), F32),
        pltpu.VMEM((CH,), I32),
        pltpu.VMEM((CH,), I32),
        pltpu.VMEM((CH,), I32),
        pltpu.VMEM((CH,), I32),
        pltpu.VMEM((CH, 32), F32),
        pltpu.VMEM((CH, 16), F32),
        pltpu.VMEM((CH, 16), F32),
        pltpu.VMEM((CH, 32), F32),
        pltpu.VMEM((CH, 16), F32),
        pltpu.VMEM((CH, 16), F32),
        pltpu.VMEM((CH, 16), F32),
        pltpu.VMEM((CH, 16), F32),
        pltpu.VMEM((CH,), I32),
        pltpu.VMEM((CH,), I32),
        pltpu.VMEM((CH, 16), F32),
        pltpu.VMEM((CH, 16), F32),
        pltpu.VMEM((CH,), I32),
        pltpu.VMEM((CH,), I32),
        pltpu.SemaphoreType.DMA,
        pltpu.SemaphoreType.DMA,
        pltpu.SemaphoreType.DMA,
        pltpu.SemaphoreType.DMA,
        pltpu.SemaphoreType.DMA,
        pltpu.SemaphoreType.DMA,
    ],
)


def _sc_conv(gidx, sidx, kv, q, ee):
    z16 = jnp.zeros((ZR, 16), F32)
    return _conv_call(gidx, sidx, kv, q, ee, z16)


def _gather_body(src_hbm, dst_hbm, vt_hbm, ct_hbm, gv_out, gc_out,
                 si0, di0, si1, di1, v0, c0, v1, c1, semb0, semb1):
    cx = lax.axis_index("c")
    s = lax.axis_index("s")
    wid = s * NCORES + cx
    base = wid * EPT

    def load_idx(i, sv, dv):
        e0 = base + i * CH
        pltpu.sync_copy(src_hbm.at[pl.ds(e0, CH)], sv)
        pltpu.sync_copy(dst_hbm.at[pl.ds(e0, CH)], dv)

    def fire(sv, dv, vb, cb, sem):
        pltpu.async_copy(vt_hbm.at[sv], vb, sem)
        pltpu.async_copy(ct_hbm.at[dv], cb, sem)

    def drain_store(i, sv, dv, vb, cb, sem):
        e0 = base + i * CH
        pltpu.make_async_copy(vt_hbm.at[sv], vb, sem).wait()
        pltpu.make_async_copy(ct_hbm.at[dv], cb, sem).wait()
        pltpu.sync_copy(vb, gv_out.at[pl.ds(e0, CH)])
        pltpu.sync_copy(cb, gc_out.at[pl.ds(e0, CH)])

    load_idx(0, si0, di0)
    fire(si0, di0, v0, c0, semb0)

    def pair(t, _):
        i0 = 2 * t
        load_idx(i0 + 1, si1, di1)
        fire(si1, di1, v1, c1, semb1)
        drain_store(i0, si0, di0, v0, c0, semb0)
        load_idx(i0 + 2, si0, di0)
        fire(si0, di0, v0, c0, semb0)
        drain_store(i0 + 1, si1, di1, v1, c1, semb1)
        return _
    lax.fori_loop(0, (NFULL - 1) // 2, pair, 0)
    drain_store(NFULL - 1, si0, di0, v0, c0, semb0)


_gather_call = pl.kernel(
    _gather_body,
    out_type=[jax.ShapeDtypeStruct((E, 16), F32),
              jax.ShapeDtypeStruct((E, 16), F32)],
    mesh=_MESH,
    compiler_params=pltpu.CompilerParams(
        needs_layout_passes=False, use_tc_tiling_on_sc=False),
    scratch_types=[
        pltpu.VMEM((CH,), I32),
        pltpu.VMEM((CH,), I32),
        pltpu.VMEM((CH,), I32),
        pltpu.VMEM((CH,), I32),
        pltpu.VMEM((CH, 16), F32),
        pltpu.VMEM((CH, 16), F32),
        pltpu.VMEM((CH, 16), F32),
        pltpu.VMEM((CH, 16), F32),
        pltpu.SemaphoreType.DMA,
        pltpu.SemaphoreType.DMA,
    ],
)


def _full(shape):
    return pl.BlockSpec(shape, lambda i: tuple(0 for _ in shape))


def _rows(blk, w):
    return pl.BlockSpec((blk, w), lambda i: (i, 0))


def _prep1_body(con_ref, var_ref, wq, bq, wk, bk, wv, bv, ws, bs,
                q_ref, kv_ref, sk_ref):
    con = con_ref[...]
    var = var_ref[...]
    q_ref[...] = wq[...] @ con + bq[...]
    k = wk[...] @ var + bk[...]
    v = wv[...] @ var + bv[...]
    kv_ref[...] = jnp.concatenate([k, v], axis=0)
    sk_ref[...] = ws[...] @ con + bs[...]


def _cols(h, blk):
    return pl.BlockSpec((h, blk), lambda i: (0, i))


def _prep1(conT, varT, p):
    return pl.pallas_call(
        _prep1_body,
        out_shape=[jax.ShapeDtypeStruct((16, NV), F32),
                   jax.ShapeDtypeStruct((32, NV), F32),
                   jax.ShapeDtypeStruct((16, NV), F32)],
    )(conT, varT,
      p["q"]["W"].T, p["q"]["b"][:, None], p["k"]["W"].T, p["k"]["b"][:, None],
      p["v"]["W"].T, p["v"]["b"][:, None], p["skip"]["W"].T, p["skip"]["b"][:, None])


def _ee_body(ec_ref, w1, b1, w2, b2, e1_ref, e2_ref):
    ec = ec_ref[...]
    e1_ref[...] = ec @ w1[...] + b1[...]
    e2_ref[...] = ec @ w2[...] + b2[...]


def _ee(edge_comb, p1, p2):
    blk = 12800
    return pl.pallas_call(
        _ee_body,
        grid=(E // blk,),
        in_specs=[_rows(blk, 7), _full((7, 16)), _full((1, 16)),
                  _full((7, 16)), _full((1, 16))],
        out_specs=[_rows(blk, 16), _rows(blk, 16)],
        out_shape=[jax.ShapeDtypeStruct((E, 16), F32),
                   jax.ShapeDtypeStruct((E, 16), F32)],
    )(edge_comb, p1["W"], p1["b"][None], p2["W"], p2["b"][None])


def _norm_agg(agg_ref, aux_ref, sk_ref):
    agg = agg_ref[0] + agg_ref[1]
    aux = aux_ref[0] + aux_ref[1]
    denom = aux[0:1, :]
    cnt = aux[1:2, :]
    return jax.nn.relu(
        agg / (denom + 1e-16) / jnp.clip(cnt, 1.0, None) + sk_ref[...])


def _combine1_body(agg_ref, aux_ref, sk_ref, con_ref,
                   wk, bk, wv, bv, c1w, c1b, c2w, c2b,
                   cl_ref, kv_ref, cc_ref):
    con_learned = _norm_agg(agg_ref, aux_ref, sk_ref)
    cl_ref[...] = con_learned
    comb = jnp.concatenate([con_learned, con_ref[...]], axis=0)
    k = wk[...] @ comb + bk[...]
    v = wv[...] @ comb + bv[...]
    kv_ref[...] = jnp.concatenate([k, v], axis=0)
    cc = jax.nn.relu(c1w[...] @ comb + c1b[...])
    cc_ref[...] = jax.nn.relu(c2w[...] @ cc + c2b[...])


def _combine1(aggT, auxT, sk1T, conT, pv, pcc1, pcc2):
    return pl.pallas_call(
        _combine1_body,
        out_shape=[jax.ShapeDtypeStruct((16, NV), F32),
                   jax.ShapeDtypeStruct((32, NV), F32),
                   jax.ShapeDtypeStruct((8, NV), F32)],
    )(aggT, auxT, sk1T, conT,
      pv["k"]["W"].T, pv["k"]["b"][:, None],
      pv["v"]["W"].T, pv["v"]["b"][:, None],
      pcc1["W"].T, pcc1["b"][:, None], pcc2["W"].T, pcc2["b"][:, None])


def _qs_body(x_ref, wq, bq, ws, bs, q_ref, sk_ref):
    x = x_ref[...]
    q_ref[...] = wq[...] @ x + bq[...]
    sk_ref[...] = ws[...] @ x + bs[...]


def _qs(xT, p):
    return pl.pallas_call(
        _qs_body,
        out_shape=[jax.ShapeDtypeStruct((16, NV), F32),
                   jax.ShapeDtypeStruct((16, NV), F32)],
    )(xT, p["q"]["W"].T, p["q"]["b"][:, None],
      p["skip"]["W"].T, p["skip"]["b"][:, None])


def _combine2_body(agg_ref, aux_ref, sk_ref, var_ref,
                   v1w, v1b, v2w, v2b, vl_ref, vc_ref):
    var_learned = _norm_agg(agg_ref, aux_ref, sk_ref)
    vl_ref[...] = var_learned
    comb = jnp.concatenate([var_learned, var_ref[...]], axis=0)
    vc = jax.nn.relu(v1w[...] @ comb + v1b[...])
    vc_ref[...] = jax.nn.relu(v2w[...] @ vc + v2b[...])


def _combine2(aggT, auxT, sk2T, varT, pvc1, pvc2):
    return pl.pallas_call(
        _combine2_body,
        out_shape=[jax.ShapeDtypeStruct((16, NV), F32),
                   jax.ShapeDtypeStruct((8, NV), F32)],
    )(aggT, auxT, sk2T, varT,
      pvc1["W"].T, pvc1["b"][:, None], pvc2["W"].T, pvc2["b"][:, None])


def _emlp_body(ec_ref, gv_ref, gc_ref, w1, b1, w2, b2, out_ref):
    e_in = jnp.concatenate(
        [ec_ref[...], gv_ref[:, 0:8], gc_ref[:, 0:8]], axis=1)
    h = jax.nn.relu(e_in @ w1[...] + b1[...])
    out_ref[...] = jax.nn.relu(h @ w2[...] + b2[...])


def _emlp(edge_comb, gv, gc, p1, p2):
    blk = 6400
    return pl.pallas_call(
        _emlp_body,
        grid=(E // blk,),
        in_specs=[_rows(blk, 7), _rows(blk, 16), _rows(blk, 16),
                  _full((23, 8)), _full((1, 8)), _full((8, 8)), _full((1, 8))],
        out_specs=_rows(blk, 8),
        out_shape=jax.ShapeDtypeStruct((E, 8), F32),
    )(edge_comb, gv, gc, p1["W"], p1["b"][None], p2["W"], p2["b"][None])


def kernel(var_lp_f, con_lp_f, lo_costs, hi_costs, def_mm, edge_lp_f_wo_ss,
           params, edge_index_var_con):
    src = edge_index_var_con[0]
    dst = edge_index_var_con[1]
    edge_comb = jnp.concatenate(
        [lo_costs[:, None], hi_costs[:, None], def_mm[:, None],
         edge_lp_f_wo_ss], axis=1)

    def unpack_aux(aux):
        return aux.reshape(2, AUXR * 8, 2).transpose(0, 2, 1)

    q1T, kv1T, sk1T = _prep1(con_lp_f.T, var_lp_f.T, params["con_conv"])
    ee1, ee2 = _ee(edge_comb, params["con_conv"]["e"], params["var_conv"]["e"])

    agg1, aux1 = _sc_conv(src, dst, kv1T.T, q1T.T, ee1)
    clT, kv2T, ccT = _combine1(
        agg1.transpose(0, 2, 1), unpack_aux(aux1), sk1T,
        con_lp_f.T, params["var_conv"], params["cc1"], params["cc2"])
    q2T, sk2T = _qs(var_lp_f.T, params["var_conv"])

    agg2, aux2 = _sc_conv(dst, src, kv2T.T, q2T.T, ee2)
    vlT, vcT = _combine2(
        agg2.transpose(0, 2, 1), unpack_aux(aux2), sk2T,
        var_lp_f.T, params["vc1"], params["vc2"])

    pad8 = jnp.zeros((8, NV), F32)
    vtab = jnp.concatenate([vcT, pad8], axis=0).T
    ctab = jnp.concatenate([ccT, pad8], axis=0).T
    gv, gc = _gather_call(src, dst, vtab, ctab)
    edge_learned = _emlp(edge_comb, gv, gc, params["em1"], params["em2"])
    return (vlT.T, clT.T, edge_learned)
